# baseline (speedup 1.0000x reference)
import functools

import numpy as np
import jax
import jax.numpy as jnp
from jax import lax
from jax.experimental import pallas as pl
from jax.experimental.pallas import tpu as pltpu
from jax.experimental.pallas import tpu_sc as plsc

B = 4
N = 8192
S = 1024
K = 32
R2 = 0.4 * 0.4
CIN = 64
C1 = 64
C2 = 128
L = 16
NWORK = 32
SPW = S * B // NWORK
G = 16
NGRP = SPW // G


def _fps_body(x_ref, o_ref, d_ref):
    xs = x_ref[0]
    ys = x_ref[1]
    zs = x_ref[2]
    d_ref[...] = jnp.full((B, 8, N // 8), 1e10, jnp.float32)
    sub_i = lax.broadcasted_iota(jnp.int32, (B, 8, N // 8), 1)
    lane_i = lax.broadcasted_iota(jnp.int32, (B, 8, N // 8), 2)
    flat_i = sub_i * (N // 8) + lane_i
    lane8 = lax.broadcasted_iota(jnp.int32, (B, 1, 8), 2)

    def step(t, far):
        onehot = flat_i == far
        cx = jnp.sum(jnp.where(onehot, xs, 0.0), axis=(1, 2), keepdims=True)
        cy = jnp.sum(jnp.where(onehot, ys, 0.0), axis=(1, 2), keepdims=True)
        cz = jnp.sum(jnp.where(onehot, zs, 0.0), axis=(1, 2), keepdims=True)
        val = jnp.where(lane8 == 0, cx, jnp.where(lane8 == 1, cy, cz))
        o_ref[:, pl.ds(t, 1), :] = val
        dx = xs - cx
        dy = ys - cy
        dz = zs - cz
        d = dx * dx + dy * dy + dz * dz
        dmin = jnp.minimum(d_ref[...], d)
        d_ref[...] = dmin
        m = jnp.max(dmin, axis=(1, 2), keepdims=True)
        newfar = jnp.min(jnp.where(dmin == m, flat_i, N), axis=(1, 2),
                         keepdims=True)
        return newfar

    lax.fori_loop(0, S, step, jnp.zeros((B, 1, 1), jnp.int32))


def _fps(xyz):
    xx = jnp.transpose(xyz, (1, 0, 2)).reshape(3, B, 8, N // 8)
    return pl.pallas_call(
        _fps_body,
        out_shape=jax.ShapeDtypeStruct((B, S, 8), jnp.float32),
        scratch_shapes=[pltpu.VMEM((B, 8, N // 8), jnp.float32)],
    )(xx)


NB2 = 512


def _k2_body(xyz8_ref, feat_ref, w1x8_ref, w1f_ref, pt_ref, sq_ref):
    xb = xyz8_ref[0]
    fb = feat_ref[0]
    ptx = lax.dot_general(xb, w1x8_ref[...], (((0,), (1,)), ((), ())),
                          preferred_element_type=jnp.float32)
    ptf = lax.dot_general(fb, w1f_ref[...], (((0,), (1,)), ((), ())),
                          preferred_element_type=jnp.float32)
    pt_ref[0] = ptx + ptf
    sq_ref[0] = jnp.sum(xb * xb, axis=0, keepdims=True)


def _k2(xyz8, features, w1x8, w1f):
    return pl.pallas_call(
        _k2_body,
        grid=(B, N // NB2),
        in_specs=[
            pl.BlockSpec((1, 8, NB2), lambda b, i: (b, 0, i)),
            pl.BlockSpec((1, CIN, NB2), lambda b, i: (b, 0, i)),
            pl.BlockSpec((C1, 8), lambda b, i: (0, 0)),
            pl.BlockSpec((C1, CIN), lambda b, i: (0, 0)),
        ],
        out_specs=[
            pl.BlockSpec((1, NB2, C1), lambda b, i: (b, i, 0)),
            pl.BlockSpec((1, 1, NB2), lambda b, i: (b, 0, i)),
        ],
        out_shape=[
            jax.ShapeDtypeStruct((B, N, C1), jnp.float32),
            jax.ShapeDtypeStruct((B, 1, N), jnp.float32),
        ],
    )(xyz8, features, w1x8, w1f)


CG = 256
NC2 = 2048
W_PER_ROW = N // L


def _k2b_body(o_ref, xyz8_ref, sq_ref, pw_ref, w_ref):
    nb = o_ref[0]
    nx = nb[:, 0:1]
    ny = nb[:, 1:2]
    nz = nb[:, 2:3]
    sqn = (nx * nx + ny * ny) + nz * nz
    xb = xyz8_ref[0]
    x = xb[0:1, :]
    y = xb[1:2, :]
    z = xb[2:3, :]
    dot = (nx * x + ny * y) + nz * z
    sqrd = (sqn + sq_ref[0]) - 2.0 * dot
    mask01 = jnp.where(sqrd <= R2, 1.0, 0.0)
    w = lax.dot_general(mask01, pw_ref[...], (((1,), (0,)), ((), ())),
                        preferred_element_type=jnp.float32)
    w_ref[0, 0] = w.astype(jnp.int32)


def _k2b(o, xyz8, sq3d, pw):
    return pl.pallas_call(
        _k2b_body,
        grid=(B, S // CG, N // NC2),
        in_specs=[
            pl.BlockSpec((1, CG, 8), lambda b, cg, nc: (b, cg, 0)),
            pl.BlockSpec((1, 8, NC2), lambda b, cg, nc: (b, 0, nc)),
            pl.BlockSpec((1, 1, NC2), lambda b, cg, nc: (b, 0, nc)),
            pl.BlockSpec((NC2, NC2 // L), lambda b, cg, nc: (0, 0)),
        ],
        out_specs=pl.BlockSpec((1, 1, CG, NC2 // L),
                               lambda b, cg, nc: (b, nc, cg, 0)),
        out_shape=jax.ShapeDtypeStruct((B, N // NC2, S, NC2 // L), jnp.int32),
    )(o, xyz8, sq3d, pw)


def _sc_body(w_h, pt_h, out_h, wv, tmpi, cidx, rows, sem, sem2):
    wid = lax.axis_index("s") * 2 + lax.axis_index("c")
    b = wid // (NWORK // B)
    sb = (wid % (NWORK // B)) * SPW

    lane = lax.broadcasted_iota(jnp.int32, (L,), 0)
    base = b * N

    def out_rows(g):
        return out_h.at[pl.ds((b * S + sb + g * G) * K, G * K)]

    def group(g, _):
        for j in range(4):
            pltpu.sync_copy(
                w_h.at[pl.ds(((b * 4 + j) * S + sb + g * G) * 128, G * 128)],
                wv.at[pl.ds(j * G * 128, G * 128)])
        rbase = (g % 2) * (G * K)
        rhalf = rows.at[pl.ds(rbase, G * K)]

        def cent(c2, _):
            def ocond(st):
                wc, cnt = st
                return jnp.logical_and(cnt < K, wc < W_PER_ROW // L)

            def obody(st):
                wc, cnt = st
                wvv = wv[pl.ds((wc // 8) * (G * 128) + c2 * 128
                               + (wc % 8) * L, L)]

                def icond(ist):
                    wrem, cnt2 = ist
                    return jnp.logical_and(jnp.any(wrem != 0), cnt2 < K)

                def ibody(ist):
                    wrem, cnt2 = ist
                    ffs = plsc.all_reduce_ffs(wrem != 0)
                    w = jnp.sum(jnp.where(lane == ffs, wrem, 0))
                    bits = jnp.bitwise_and(jnp.right_shift(w, lane), 1)
                    m = bits != 0
                    pref = plsc.cumsum(bits)
                    dest = cnt2 + pref - 1
                    pos = (wc * L + ffs) * L + lane
                    plsc.store_scatter(tmpi, [dest], pos, mask=m)
                    return (jnp.where(lane == ffs, 0, wrem),
                            cnt2 + jnp.sum(bits))

                _, cnt = lax.while_loop(icond, ibody, (wvv, cnt))
                return wc + 1, cnt

            _, cntf = lax.while_loop(ocond, obody, (jnp.int32(0), jnp.int32(0)))
            f16 = tmpi[pl.ds(0, L)]
            s16 = tmpi[pl.ds(L, L)]
            first = jnp.sum(jnp.where(lane == 0, f16, 0))
            v0 = jnp.where(lane < cntf, f16, first) + base
            v1 = jnp.where(lane + L < cntf, s16, first) + base
            cidx[pl.ds(c2 * K, L)] = v0
            cidx[pl.ds(c2 * K + L, L)] = v1
            return 0

        lax.fori_loop(0, G, cent, 0)
        @pl.when(g >= 2)
        def _():
            pltpu.make_async_copy(rhalf, out_rows(g - 2), sem2).wait()
        cps = []
        for q in range(4):
            cps.append(pltpu.async_copy(
                pt_h.at[cidx.at[pl.ds(q * 128, 128)]],
                rows.at[pl.ds(rbase + q * 128, 128)], sem))
        for cp in cps:
            cp.wait()
        pltpu.async_copy(rhalf, out_rows(g), sem2)
        return 0

    lax.fori_loop(0, NGRP, group, 0)
    pltpu.make_async_copy(rows.at[pl.ds(0, G * K)], out_rows(NGRP - 2), sem2).wait()
    pltpu.make_async_copy(rows.at[pl.ds(G * K, G * K)], out_rows(NGRP - 1), sem2).wait()


def _sc_ballq_gather(w_flat, pt_flat):
    mesh = plsc.VectorSubcoreMesh(core_axis_name="c", subcore_axis_name="s")
    f = pl.kernel(
        _sc_body,
        out_type=jax.ShapeDtypeStruct((B * S * K, C1), jnp.float32),
        mesh=mesh,
        compiler_params=pltpu.CompilerParams(
            needs_layout_passes=False, use_tc_tiling_on_sc=False),
        scratch_types=[
            pltpu.VMEM((G * W_PER_ROW,), jnp.int32),
            pltpu.VMEM((96,), jnp.int32),
            pltpu.VMEM((G * K,), jnp.int32),
            pltpu.VMEM((2 * G * K, C1), jnp.float32),
            pltpu.SemaphoreType.DMA,
            pltpu.SemaphoreType.DMA,
        ],
    )
    return f(w_flat, pt_flat)


SB4 = 64


def _k4_body(h_ref, o8_ref, w1x8_ref, b1_ref, w2p_ref, b2p_ref, out_ref):
    hb = h_ref[...]
    nb = o8_ref[0]
    c1 = lax.dot_general(nb, w1x8_ref[...], (((1,), (1,)), ((), ())),
                         preferred_element_type=jnp.float32)
    c1p = jnp.concatenate([c1, c1], axis=1)
    b1p = jnp.concatenate([b1_ref[...], b1_ref[...]], axis=1)
    h3 = hb.reshape(SB4, K // 2, 2 * C1)
    h1 = jnp.maximum(h3 - c1p[:, None, :] + b1p[None, :, :], 0.0)
    h2 = lax.dot_general(h1.reshape(SB4 * K // 2, 2 * C1), w2p_ref[...],
                         (((1,), (0,)), ((), ())),
                         preferred_element_type=jnp.float32)
    h2 = jnp.maximum(h2 + b2p_ref[...], 0.0)
    mk = jnp.max(h2.reshape(SB4, K // 2, 2 * C2), axis=1)
    out_ref[0] = jnp.maximum(mk[:, :C2], mk[:, C2:])


def _k4(h2d, o, w1x8, b1r, w2p, b2p):
    return pl.pallas_call(
        _k4_body,
        grid=(B, S // SB4),
        in_specs=[
            pl.BlockSpec((SB4 * K // 2, 2 * C1),
                         lambda b, i: (b * (S // SB4) + i, 0)),
            pl.BlockSpec((1, SB4, 8), lambda b, i: (b, i, 0)),
            pl.BlockSpec((C1, 8), lambda b, i: (0, 0)),
            pl.BlockSpec((1, C1), lambda b, i: (0, 0)),
            pl.BlockSpec((2 * C1, 2 * C2), lambda b, i: (0, 0)),
            pl.BlockSpec((1, 2 * C2), lambda b, i: (0, 0)),
        ],
        out_specs=pl.BlockSpec((1, SB4, C2), lambda b, i: (b, i, 0)),
        out_shape=jax.ShapeDtypeStruct((B, S, C2), jnp.float32),
    )(h2d, o, w1x8, b1r, w2p, b2p)


def kernel(xyz, features, W1, b1, W2, b2):
    w1x8 = jnp.concatenate([W1[:, :3], jnp.zeros((C1, 5), jnp.float32)], axis=1)
    w1f = W1[:, 3:]
    xyz8 = jnp.concatenate([xyz, jnp.zeros((B, 5, N), jnp.float32)], axis=1)

    o = _fps(xyz)
    new_xyz = jnp.transpose(o[:, :, :3], (0, 2, 1))

    pt, sq3d = _k2(xyz8, features, w1x8, w1f)

    pw = jnp.asarray(
        np.eye(NC2 // L, dtype=np.float32).repeat(L, axis=0)
        * np.tile((2.0 ** np.arange(L, dtype=np.float32)), NC2 // L)[:, None])
    w = _k2b(o, xyz8, sq3d, pw)

    h = _sc_ballq_gather(w.reshape(B * S * W_PER_ROW),
                         pt.reshape(B * N, C1))

    w2p = jnp.zeros((2 * C1, 2 * C2), jnp.float32)
    w2p = w2p.at[:C1, :C2].set(W2.T).at[C1:, C2:].set(W2.T)
    b2p = jnp.concatenate([b2, b2]).reshape(1, 2 * C2)
    out_t = _k4(h.reshape(B * S * K // 2, 2 * C1), o, w1x8,
                b1.reshape(1, C1), w2p, b2p)
    new_features = jnp.transpose(out_t, (0, 2, 1))
    return (new_xyz, new_features)


# speedup vs baseline: 1.0131x; 1.0131x over previous
import functools

import numpy as np
import jax
import jax.numpy as jnp
from jax import lax
from jax.experimental import pallas as pl
from jax.experimental.pallas import tpu as pltpu
from jax.experimental.pallas import tpu_sc as plsc

B = 4
N = 8192
S = 1024
K = 32
R2 = 0.4 * 0.4
CIN = 64
C1 = 64
C2 = 128
L = 16
NWORK = 32
SPW = S * B // NWORK
G = 16
NGRP = SPW // G


def _fps_body(x_ref, o_ref, d_ref):
    xs = x_ref[0]
    ys = x_ref[1]
    zs = x_ref[2]
    d_ref[...] = jnp.full((B, 8, N // 8), 1e10, jnp.float32)
    sub_i = lax.broadcasted_iota(jnp.int32, (B, 8, N // 8), 1)
    lane_i = lax.broadcasted_iota(jnp.int32, (B, 8, N // 8), 2)
    flat_i = sub_i * (N // 8) + lane_i
    lane8 = lax.broadcasted_iota(jnp.int32, (B, 1, 8), 2)

    def step(t, far):
        onehot = flat_i == far
        cx = jnp.sum(jnp.where(onehot, xs, 0.0), axis=(1, 2), keepdims=True)
        cy = jnp.sum(jnp.where(onehot, ys, 0.0), axis=(1, 2), keepdims=True)
        cz = jnp.sum(jnp.where(onehot, zs, 0.0), axis=(1, 2), keepdims=True)
        val = jnp.where(lane8 == 0, cx, jnp.where(lane8 == 1, cy, cz))
        o_ref[:, pl.ds(t, 1), :] = val
        dx = xs - cx
        dy = ys - cy
        dz = zs - cz
        d = dx * dx + dy * dy + dz * dz
        dmin = jnp.minimum(d_ref[...], d)
        d_ref[...] = dmin
        m = jnp.max(dmin, axis=(1, 2), keepdims=True)
        newfar = jnp.min(jnp.where(dmin == m, flat_i, N), axis=(1, 2),
                         keepdims=True)
        return newfar

    lax.fori_loop(0, S, step, jnp.zeros((B, 1, 1), jnp.int32))


def _fps(xyz):
    xx = jnp.transpose(xyz, (1, 0, 2)).reshape(3, B, 8, N // 8)
    return pl.pallas_call(
        _fps_body,
        out_shape=jax.ShapeDtypeStruct((B, S, 8), jnp.float32),
        scratch_shapes=[pltpu.VMEM((B, 8, N // 8), jnp.float32)],
    )(xx)


NB2 = 512


def _k2_body(xyz8_ref, feat_ref, w1x8_ref, w1f_ref, pt_ref, sq_ref):
    xb = xyz8_ref[0]
    fb = feat_ref[0]
    ptx = lax.dot_general(xb, w1x8_ref[...], (((0,), (1,)), ((), ())),
                          preferred_element_type=jnp.float32)
    ptf = lax.dot_general(fb, w1f_ref[...], (((0,), (1,)), ((), ())),
                          preferred_element_type=jnp.float32)
    pt_ref[0] = ptx + ptf
    sq_ref[0] = jnp.sum(xb * xb, axis=0, keepdims=True)


def _k2(xyz8, features, w1x8, w1f):
    return pl.pallas_call(
        _k2_body,
        grid=(B, N // NB2),
        in_specs=[
            pl.BlockSpec((1, 8, NB2), lambda b, i: (b, 0, i)),
            pl.BlockSpec((1, CIN, NB2), lambda b, i: (b, 0, i)),
            pl.BlockSpec((C1, 8), lambda b, i: (0, 0)),
            pl.BlockSpec((C1, CIN), lambda b, i: (0, 0)),
        ],
        out_specs=[
            pl.BlockSpec((1, NB2, C1), lambda b, i: (b, i, 0)),
            pl.BlockSpec((1, 1, NB2), lambda b, i: (b, 0, i)),
        ],
        out_shape=[
            jax.ShapeDtypeStruct((B, N, C1), jnp.float32),
            jax.ShapeDtypeStruct((B, 1, N), jnp.float32),
        ],
    )(xyz8, features, w1x8, w1f)


CG = 256
NC2 = 2048
W_PER_ROW = N // L


def _k2b_body(o_ref, xyz8_ref, sq_ref, pw_ref, w_ref):
    nb = o_ref[0]
    nx = nb[:, 0:1]
    ny = nb[:, 1:2]
    nz = nb[:, 2:3]
    sqn = (nx * nx + ny * ny) + nz * nz
    xb = xyz8_ref[0]
    x = xb[0:1, :]
    y = xb[1:2, :]
    z = xb[2:3, :]
    dot = (nx * x + ny * y) + nz * z
    sqrd = (sqn + sq_ref[0]) - 2.0 * dot
    mask01 = jnp.where(sqrd <= R2, 1.0, 0.0)
    w = lax.dot_general(mask01, pw_ref[...], (((1,), (0,)), ((), ())),
                        preferred_element_type=jnp.float32)
    w_ref[0] = w.astype(jnp.int32)


def _k2b(o, xyz8, sq3d, pw):
    return pl.pallas_call(
        _k2b_body,
        grid=(B, S // CG, N // NC2),
        in_specs=[
            pl.BlockSpec((1, CG, 8), lambda b, cg, nc: (b, cg, 0)),
            pl.BlockSpec((1, 8, NC2), lambda b, cg, nc: (b, 0, nc)),
            pl.BlockSpec((1, 1, NC2), lambda b, cg, nc: (b, 0, nc)),
            pl.BlockSpec((NC2, NC2 // L), lambda b, cg, nc: (0, 0)),
        ],
        out_specs=pl.BlockSpec((1, CG, NC2 // L), lambda b, cg, nc: (b, cg, nc)),
        out_shape=jax.ShapeDtypeStruct((B, S, W_PER_ROW), jnp.int32),
    )(o, xyz8, sq3d, pw)


def _sc_body(w_h, pt_h, out_h, wv, tmpi, cidx, rows, sem, sem2):
    wid = lax.axis_index("s") * 2 + lax.axis_index("c")
    b = wid // (NWORK // B)
    sb = (wid % (NWORK // B)) * SPW

    lane = lax.broadcasted_iota(jnp.int32, (L,), 0)
    base = b * N

    def out_rows(g):
        return out_h.at[pl.ds((b * S + sb + g * G) * K, G * K)]

    def group(g, _):
        wstart = (b * S + sb + g * G) * W_PER_ROW
        pltpu.sync_copy(w_h.at[pl.ds(wstart, G * W_PER_ROW)], wv)
        rbase = (g % 2) * (G * K)
        rhalf = rows.at[pl.ds(rbase, G * K)]

        def cent(c2, _):
            woff = c2 * W_PER_ROW

            def ocond(st):
                wc, cnt = st
                return jnp.logical_and(cnt < K, wc < W_PER_ROW // L)

            def obody(st):
                wc, cnt = st
                wvv = wv[pl.ds(woff + wc * L, L)]

                def icond(ist):
                    wrem, cnt2 = ist
                    return jnp.logical_and(jnp.any(wrem != 0), cnt2 < K)

                def ibody(ist):
                    wrem, cnt2 = ist
                    ffs = plsc.all_reduce_ffs(wrem != 0)
                    w = jnp.sum(jnp.where(lane == ffs, wrem, 0))
                    bits = jnp.bitwise_and(jnp.right_shift(w, lane), 1)
                    m = bits != 0
                    pref = plsc.cumsum(bits)
                    dest = cnt2 + pref - 1
                    pos = (wc * L + ffs) * L + lane
                    plsc.store_scatter(tmpi, [dest], pos, mask=m)
                    return (jnp.where(lane == ffs, 0, wrem),
                            cnt2 + jnp.sum(bits))

                _, cnt = lax.while_loop(icond, ibody, (wvv, cnt))
                return wc + 1, cnt

            _, cntf = lax.while_loop(ocond, obody, (jnp.int32(0), jnp.int32(0)))
            f16 = tmpi[pl.ds(0, L)]
            s16 = tmpi[pl.ds(L, L)]
            first = jnp.sum(jnp.where(lane == 0, f16, 0))
            v0 = jnp.where(lane < cntf, f16, first) + base
            v1 = jnp.where(lane + L < cntf, s16, first) + base
            cidx[pl.ds(c2 * K, L)] = v0
            cidx[pl.ds(c2 * K + L, L)] = v1
            return 0

        lax.fori_loop(0, G, cent, 0)
        @pl.when(g >= 2)
        def _():
            pltpu.make_async_copy(rhalf, out_rows(g - 2), sem2).wait()
        cps = []
        for q in range(4):
            cps.append(pltpu.async_copy(
                pt_h.at[cidx.at[pl.ds(q * 128, 128)]],
                rows.at[pl.ds(rbase + q * 128, 128)], sem))
        for cp in cps:
            cp.wait()
        pltpu.async_copy(rhalf, out_rows(g), sem2)
        return 0

    lax.fori_loop(0, NGRP, group, 0)
    pltpu.make_async_copy(rows.at[pl.ds(0, G * K)], out_rows(NGRP - 2), sem2).wait()
    pltpu.make_async_copy(rows.at[pl.ds(G * K, G * K)], out_rows(NGRP - 1), sem2).wait()


def _sc_ballq_gather(w_flat, pt_flat):
    mesh = plsc.VectorSubcoreMesh(core_axis_name="c", subcore_axis_name="s")
    f = pl.kernel(
        _sc_body,
        out_type=jax.ShapeDtypeStruct((B * S * K, C1), jnp.float32),
        mesh=mesh,
        compiler_params=pltpu.CompilerParams(
            needs_layout_passes=False, use_tc_tiling_on_sc=False),
        scratch_types=[
            pltpu.VMEM((G * W_PER_ROW,), jnp.int32),
            pltpu.VMEM((96,), jnp.int32),
            pltpu.VMEM((G * K,), jnp.int32),
            pltpu.VMEM((2 * G * K, C1), jnp.float32),
            pltpu.SemaphoreType.DMA,
            pltpu.SemaphoreType.DMA,
        ],
    )
    return f(w_flat, pt_flat)


SB4 = 64


def _k4_body(h_ref, o8_ref, w1x8_ref, b1_ref, w2p_ref, b2p_ref, out_ref):
    hb = h_ref[...]
    nb = o8_ref[0]
    c1 = lax.dot_general(nb, w1x8_ref[...], (((1,), (1,)), ((), ())),
                         preferred_element_type=jnp.float32)
    c1p = jnp.concatenate([c1, c1], axis=1)
    b1p = jnp.concatenate([b1_ref[...], b1_ref[...]], axis=1)
    h3 = hb.reshape(SB4, K // 2, 2 * C1)
    h1 = jnp.maximum(h3 - c1p[:, None, :] + b1p[None, :, :], 0.0)
    h2 = lax.dot_general(h1.reshape(SB4 * K // 2, 2 * C1), w2p_ref[...],
                         (((1,), (0,)), ((), ())),
                         preferred_element_type=jnp.float32)
    h2 = jnp.maximum(h2 + b2p_ref[...], 0.0)
    mk = jnp.max(h2.reshape(SB4, K // 2, 2 * C2), axis=1)
    out_ref[0] = jnp.maximum(mk[:, :C2], mk[:, C2:])


def _k4(h2d, o, w1x8, b1r, w2p, b2p):
    return pl.pallas_call(
        _k4_body,
        grid=(B, S // SB4),
        in_specs=[
            pl.BlockSpec((SB4 * K // 2, 2 * C1),
                         lambda b, i: (b * (S // SB4) + i, 0)),
            pl.BlockSpec((1, SB4, 8), lambda b, i: (b, i, 0)),
            pl.BlockSpec((C1, 8), lambda b, i: (0, 0)),
            pl.BlockSpec((1, C1), lambda b, i: (0, 0)),
            pl.BlockSpec((2 * C1, 2 * C2), lambda b, i: (0, 0)),
            pl.BlockSpec((1, 2 * C2), lambda b, i: (0, 0)),
        ],
        out_specs=pl.BlockSpec((1, SB4, C2), lambda b, i: (b, i, 0)),
        out_shape=jax.ShapeDtypeStruct((B, S, C2), jnp.float32),
    )(h2d, o, w1x8, b1r, w2p, b2p)


def kernel(xyz, features, W1, b1, W2, b2):
    w1x8 = jnp.concatenate([W1[:, :3], jnp.zeros((C1, 5), jnp.float32)], axis=1)
    w1f = W1[:, 3:]
    xyz8 = jnp.concatenate([xyz, jnp.zeros((B, 5, N), jnp.float32)], axis=1)

    o = _fps(xyz)
    new_xyz = jnp.transpose(o[:, :, :3], (0, 2, 1))

    pt, sq3d = _k2(xyz8, features, w1x8, w1f)

    pw = jnp.asarray(
        np.eye(NC2 // L, dtype=np.float32).repeat(L, axis=0)
        * np.tile((2.0 ** np.arange(L, dtype=np.float32)), NC2 // L)[:, None])
    w = _k2b(o, xyz8, sq3d, pw)

    h = _sc_ballq_gather(w.reshape(B * S * W_PER_ROW),
                         pt.reshape(B * N, C1))

    w2p = jnp.zeros((2 * C1, 2 * C2), jnp.float32)
    w2p = w2p.at[:C1, :C2].set(W2.T).at[C1:, C2:].set(W2.T)
    b2p = jnp.concatenate([b2, b2]).reshape(1, 2 * C2)
    out_t = _k4(h.reshape(B * S * K // 2, 2 * C1), o, w1x8,
                b1.reshape(1, C1), w2p, b2p)
    new_features = jnp.transpose(out_t, (0, 2, 1))
    return (new_xyz, new_features)
